# Initial kernel scaffold; baseline (speedup 1.0000x reference)
import jax
import jax.numpy as jnp
from jax.experimental import pallas as pl


def kernel(hidden_states, attention_mask, position_ids, ln1_w, ln1_b, wq, bq, wk, bk, wv, bv, wo, bo, ln2_w, ln2_b, w_router, w1, w3, w2):
    raise NotImplementedError("write your pallas kernel here")


# speedup vs baseline: 1.2905x; 1.2905x over previous
import functools

import jax
import jax.numpy as jnp
import numpy as np
from jax.experimental import pallas as pl
from jax.experimental.pallas import tpu as pltpu

S, D = 2048, 1024
H, KVH, HD = 16, 8, 64
HHD = HD // 2
E, FF = 8, 2048
THETA = 10000.0
EPS = 1e-5
RB = 256
NRB = S // RB

_INTERPRET = False


def _ln(x, w, b):
    mu = jnp.mean(x, axis=-1, keepdims=True)
    xc = x - mu
    var = jnp.mean(xc * xc, axis=-1, keepdims=True)
    return xc * jax.lax.rsqrt(var + EPS) * w + b


def _qkv_body(x_ref, cos_ref, sin_ref, ln1w_ref, ln1b_ref, wq_ref, bq_ref,
              wk_ref, bk_ref, wv_ref, bv_ref, q_ref, k_ref, v_ref):
    x = x_ref[...]
    h = _ln(x, ln1w_ref[...], ln1b_ref[...])
    q = jnp.dot(h, wq_ref[...], preferred_element_type=jnp.float32) + bq_ref[...]
    k = jnp.dot(h, wk_ref[...], preferred_element_type=jnp.float32) + bk_ref[...]
    v = jnp.dot(h, wv_ref[...], preferred_element_type=jnp.float32) + bv_ref[...]
    cos = cos_ref[...]
    sin = sin_ref[...]
    cq = jnp.concatenate([cos] * H, axis=1)
    sq = jnp.concatenate([sin] * H, axis=1)
    ck = jnp.concatenate([cos] * KVH, axis=1)
    sk = jnp.concatenate([sin] * KVH, axis=1)
    q1, q2 = q[:, :H * HHD], q[:, H * HHD:]
    k1, k2 = k[:, :KVH * HHD], k[:, KVH * HHD:]
    q_ref[...] = jnp.concatenate([q1 * cq - q2 * sq, q2 * cq + q1 * sq], axis=1)
    k_ref[...] = jnp.concatenate([k1 * ck - k2 * sk, k2 * ck + k1 * sk], axis=1)
    v_ref[...] = v


def _attn_body(q_ref, k_ref, v_ref, o_ref):
    qi = pl.program_id(0)
    q = q_ref[...]
    k = k_ref[...]
    v = v_ref[...]
    rows = qi * RB + jax.lax.broadcasted_iota(jnp.int32, (RB, S), 0)
    cols = jax.lax.broadcasted_iota(jnp.int32, (RB, S), 1)
    causal = cols <= rows
    for h in range(H):
        kv = h // (H // KVH)
        q1 = q[:, h * HHD:(h + 1) * HHD]
        q2 = q[:, H * HHD + h * HHD: H * HHD + (h + 1) * HHD]
        k1 = k[:, kv * HHD:(kv + 1) * HHD]
        k2 = k[:, KVH * HHD + kv * HHD: KVH * HHD + (kv + 1) * HHD]
        s = jax.lax.dot_general(q1, k1, (((1,), (1,)), ((), ())),
                                preferred_element_type=jnp.float32)
        s = s + jax.lax.dot_general(q2, k2, (((1,), (1,)), ((), ())),
                                    preferred_element_type=jnp.float32)
        s = jnp.where(causal, s * 0.125, jnp.float32(-1e30))
        m = jnp.max(s, axis=1, keepdims=True)
        p = jnp.exp(s - m)
        l = jnp.sum(p, axis=1, keepdims=True)
        vh = v[:, kv * HD:(kv + 1) * HD]
        o = jnp.dot(p, vh, preferred_element_type=jnp.float32) / l
        o_ref[:, h * HD:(h + 1) * HD] = o


def _post_body(x_ref, attn_ref, wo_ref, bo_ref, ln2w_ref, ln2b_ref, wr_ref,
               x2_ref, h2_ref, g_ref):
    x2 = x_ref[...] + jnp.dot(attn_ref[...], wo_ref[...],
                              preferred_element_type=jnp.float32) + bo_ref[...]
    h2 = _ln(x2, ln2w_ref[...], ln2b_ref[...])
    logits128 = jnp.dot(h2, wr_ref[...], preferred_element_type=jnp.float32)
    logits = logits128[:, :E]
    m = jnp.max(logits, axis=1, keepdims=True)
    p = jnp.exp(logits - m)
    rp = p / jnp.sum(p, axis=1, keepdims=True)
    iota = jax.lax.broadcasted_iota(jnp.int32, (RB, E), 1)
    v1 = jnp.max(rp, axis=1, keepdims=True)
    i1 = jnp.min(jnp.where(rp == v1, iota, E), axis=1, keepdims=True)
    rp2 = jnp.where(iota == i1, -1.0, rp)
    v2 = jnp.max(rp2, axis=1, keepdims=True)
    i2 = jnp.min(jnp.where(rp2 == v2, iota, E), axis=1, keepdims=True)
    denom = v1 + v2
    iota128 = jax.lax.broadcasted_iota(jnp.int32, (RB, 128), 1)
    g = (jnp.where(iota128 == i1, v1, 0.0)
         + jnp.where(iota128 == i2, v2, 0.0)) / denom
    x2_ref[...] = x2
    h2_ref[...] = h2
    g_ref[...] = g


def _moe_body(h2_ref, g_ref, x2_ref, w1_ref, w3_ref, w2_ref, out_ref, acc_ref):
    e = pl.program_id(0)
    si = pl.program_id(1)
    rows = pl.ds(si * RB, RB)
    h2 = h2_ref[...].astype(jnp.bfloat16)
    a = jnp.dot(h2, w1_ref[0], preferred_element_type=jnp.float32)
    b = jnp.dot(h2, w3_ref[0], preferred_element_type=jnp.float32)
    act = (a * jax.nn.sigmoid(a) * b).astype(jnp.bfloat16)
    y = jnp.dot(act, w2_ref[0], preferred_element_type=jnp.float32)
    iota128 = jax.lax.broadcasted_iota(jnp.int32, (RB, 128), 1)
    ge = jnp.sum(jnp.where(iota128 == e, g_ref[...], 0.0), axis=1, keepdims=True)
    contrib = ge * y

    @pl.when(e == 0)
    def _():
        acc_ref[rows, :] = contrib

    @pl.when(e > 0)
    def _():
        acc_ref[rows, :] = acc_ref[rows, :] + contrib

    @pl.when(e == E - 1)
    def _():
        out_ref[...] = acc_ref[rows, :] + x2_ref[...]


def _split_halves(w):
    g = w.shape[-1] // HD
    w3d = w.reshape(*w.shape[:-1], g, HD)
    return jnp.concatenate(
        [w3d[..., :HHD].reshape(*w.shape[:-1], g * HHD),
         w3d[..., HHD:].reshape(*w.shape[:-1], g * HHD)], axis=-1)


def kernel(hidden_states, attention_mask, position_ids, ln1_w, ln1_b, wq, bq,
           wk, bk, wv, bv, wo, bo, ln2_w, ln2_b, w_router, w1, w3, w2):
    x = hidden_states.reshape(S, D)
    pos = position_ids.reshape(S).astype(jnp.float32)
    inv = jnp.asarray(1.0 / (THETA ** (np.arange(0, HD, 2) / HD)), jnp.float32)
    ang = pos[:, None] * inv[None, :]
    cos = jnp.cos(ang)
    sin = jnp.sin(ang)

    wq_p = _split_halves(wq)
    bq_p = _split_halves(bq[None, :])[0]
    wk_p = _split_halves(wk)
    bk_p = _split_halves(bk[None, :])[0]
    wr_pad = jnp.pad(w_router, ((0, 0), (0, 128 - E)))

    row_spec = pl.BlockSpec((RB, D), lambda i: (i, 0))
    full = lambda *shape: pl.BlockSpec(shape, lambda *i: (0,) * len(shape))

    qf, kf, vf = pl.pallas_call(
        _qkv_body,
        grid=(NRB,),
        in_specs=[
            row_spec,
            pl.BlockSpec((RB, HHD), lambda i: (i, 0)),
            pl.BlockSpec((RB, HHD), lambda i: (i, 0)),
            full(D), full(D),
            full(D, H * HD), full(H * HD),
            full(D, KVH * HD), full(KVH * HD),
            full(D, KVH * HD), full(KVH * HD),
        ],
        out_specs=[
            pl.BlockSpec((RB, H * HD), lambda i: (i, 0)),
            pl.BlockSpec((RB, KVH * HD), lambda i: (i, 0)),
            pl.BlockSpec((RB, KVH * HD), lambda i: (i, 0)),
        ],
        out_shape=[
            jax.ShapeDtypeStruct((S, H * HD), jnp.float32),
            jax.ShapeDtypeStruct((S, KVH * HD), jnp.float32),
            jax.ShapeDtypeStruct((S, KVH * HD), jnp.float32),
        ],
        interpret=_INTERPRET,
    )(x, cos, sin, ln1_w, ln1_b, wq_p, bq_p, wk_p, bk_p, wv, bv)

    attn = pl.pallas_call(
        _attn_body,
        grid=(NRB,),
        in_specs=[
            pl.BlockSpec((RB, H * HD), lambda i: (i, 0)),
            full(S, KVH * HD),
            full(S, KVH * HD),
        ],
        out_specs=pl.BlockSpec((RB, H * HD), lambda i: (i, 0)),
        out_shape=jax.ShapeDtypeStruct((S, H * HD), jnp.float32),
        interpret=_INTERPRET,
    )(qf, kf, vf)

    x2, h2, gates = pl.pallas_call(
        _post_body,
        grid=(NRB,),
        in_specs=[
            row_spec,
            pl.BlockSpec((RB, H * HD), lambda i: (i, 0)),
            full(H * HD, D), full(D),
            full(D), full(D),
            full(D, 128),
        ],
        out_specs=[row_spec, row_spec, pl.BlockSpec((RB, 128), lambda i: (i, 0))],
        out_shape=[
            jax.ShapeDtypeStruct((S, D), jnp.float32),
            jax.ShapeDtypeStruct((S, D), jnp.float32),
            jax.ShapeDtypeStruct((S, 128), jnp.float32),
        ],
        interpret=_INTERPRET,
    )(x, attn, wo, bo, ln2_w, ln2_b, wr_pad)

    w1b = w1.astype(jnp.bfloat16)
    w3b = w3.astype(jnp.bfloat16)
    w2b = w2.astype(jnp.bfloat16)

    out = pl.pallas_call(
        _moe_body,
        grid=(E, NRB),
        in_specs=[
            pl.BlockSpec((RB, D), lambda e, s: (s, 0)),
            pl.BlockSpec((RB, 128), lambda e, s: (s, 0)),
            pl.BlockSpec((RB, D), lambda e, s: (s, 0)),
            pl.BlockSpec((1, D, FF), lambda e, s: (e, 0, 0)),
            pl.BlockSpec((1, D, FF), lambda e, s: (e, 0, 0)),
            pl.BlockSpec((1, FF, D), lambda e, s: (e, 0, 0)),
        ],
        out_specs=pl.BlockSpec(
            (RB, D), lambda e, s: (jnp.where(e == E - 1, s, 0), 0)),
        out_shape=jax.ShapeDtypeStruct((S, D), jnp.float32),
        scratch_shapes=[pltpu.VMEM((S, D), jnp.float32)],
        interpret=_INTERPRET,
    )(h2, gates, x2, w1b, w3b, w2b)

    return out.reshape(1, S, D)


# speedup vs baseline: 1.2953x; 1.0037x over previous
import functools

import jax
import jax.numpy as jnp
import numpy as np
from jax.experimental import pallas as pl
from jax.experimental.pallas import tpu as pltpu

S, D = 2048, 1024
H, KVH, HD = 16, 8, 64
HHD = HD // 2
E, FF = 8, 2048
THETA = 10000.0
EPS = 1e-5
RB = 256
NRB = S // RB

_INTERPRET = False


def _ln(x, w, b):
    mu = jnp.mean(x, axis=-1, keepdims=True)
    xc = x - mu
    var = jnp.mean(xc * xc, axis=-1, keepdims=True)
    return xc * jax.lax.rsqrt(var + EPS) * w + b


def _qkv_body(x_ref, cos_ref, sin_ref, ln1w_ref, ln1b_ref, wq_ref, bq_ref,
              wk_ref, bk_ref, wv_ref, bv_ref, q_ref, k_ref, v_ref):
    x = x_ref[...]
    h = _ln(x, ln1w_ref[...], ln1b_ref[...]).astype(jnp.bfloat16)
    q = jnp.dot(h, wq_ref[...], preferred_element_type=jnp.float32) + bq_ref[...]
    k = jnp.dot(h, wk_ref[...], preferred_element_type=jnp.float32) + bk_ref[...]
    v = jnp.dot(h, wv_ref[...], preferred_element_type=jnp.float32) + bv_ref[...]
    cos = cos_ref[...]
    sin = sin_ref[...]
    cq = jnp.concatenate([cos] * H, axis=1)
    sq = jnp.concatenate([sin] * H, axis=1)
    ck = jnp.concatenate([cos] * KVH, axis=1)
    sk = jnp.concatenate([sin] * KVH, axis=1)
    q1, q2 = q[:, :H * HHD], q[:, H * HHD:]
    k1, k2 = k[:, :KVH * HHD], k[:, KVH * HHD:]
    q_ref[...] = jnp.concatenate(
        [q1 * cq - q2 * sq, q2 * cq + q1 * sq], axis=1).astype(jnp.bfloat16)
    k_ref[...] = jnp.concatenate(
        [k1 * ck - k2 * sk, k2 * ck + k1 * sk], axis=1).astype(jnp.bfloat16)
    v_ref[...] = v.astype(jnp.bfloat16)


def _attn_body(q_ref, k_ref, v_ref, o_ref):
    qi = pl.program_id(0)
    q = q_ref[...]
    k = k_ref[...]
    v = v_ref[...]
    rows = qi * RB + jax.lax.broadcasted_iota(jnp.int32, (RB, S), 0)
    cols = jax.lax.broadcasted_iota(jnp.int32, (RB, S), 1)
    causal = cols <= rows
    for h in range(H):
        kv = h // (H // KVH)
        q1 = q[:, h * HHD:(h + 1) * HHD]
        q2 = q[:, H * HHD + h * HHD: H * HHD + (h + 1) * HHD]
        k1 = k[:, kv * HHD:(kv + 1) * HHD]
        k2 = k[:, KVH * HHD + kv * HHD: KVH * HHD + (kv + 1) * HHD]
        s = jax.lax.dot_general(q1, k1, (((1,), (1,)), ((), ())),
                                preferred_element_type=jnp.float32)
        s = s + jax.lax.dot_general(q2, k2, (((1,), (1,)), ((), ())),
                                    preferred_element_type=jnp.float32)
        s = jnp.where(causal, s * 0.125, jnp.float32(-1e30))
        m = jnp.max(s, axis=1, keepdims=True)
        p = jnp.exp(s - m)
        l = jnp.sum(p, axis=1, keepdims=True)
        vh = v[:, kv * HD:(kv + 1) * HD]
        o = jnp.dot(p.astype(jnp.bfloat16), vh,
                    preferred_element_type=jnp.float32) / l
        o_ref[:, h * HD:(h + 1) * HD] = o.astype(jnp.bfloat16)


def _post_body(x_ref, attn_ref, wo_ref, bo_ref, ln2w_ref, ln2b_ref, wr_ref,
               x2_ref, h2_ref, g_ref):
    x2 = x_ref[...] + jnp.dot(attn_ref[...], wo_ref[...],
                              preferred_element_type=jnp.float32) + bo_ref[...]
    h2 = _ln(x2, ln2w_ref[...], ln2b_ref[...])
    logits128 = jnp.dot(h2, wr_ref[...], preferred_element_type=jnp.float32)
    logits = logits128[:, :E]
    m = jnp.max(logits, axis=1, keepdims=True)
    p = jnp.exp(logits - m)
    rp = p / jnp.sum(p, axis=1, keepdims=True)
    iota = jax.lax.broadcasted_iota(jnp.int32, (RB, E), 1)
    v1 = jnp.max(rp, axis=1, keepdims=True)
    i1 = jnp.min(jnp.where(rp == v1, iota, E), axis=1, keepdims=True)
    rp2 = jnp.where(iota == i1, -1.0, rp)
    v2 = jnp.max(rp2, axis=1, keepdims=True)
    i2 = jnp.min(jnp.where(rp2 == v2, iota, E), axis=1, keepdims=True)
    denom = v1 + v2
    iota128 = jax.lax.broadcasted_iota(jnp.int32, (RB, 128), 1)
    g = (jnp.where(iota128 == i1, v1, 0.0)
         + jnp.where(iota128 == i2, v2, 0.0)) / denom
    x2_ref[...] = x2
    h2_ref[...] = h2
    g_ref[...] = g


def _moe_body(h2_ref, g_ref, x2_ref, w1_ref, w3_ref, w2_ref, out_ref, acc_ref):
    e = pl.program_id(0)
    si = pl.program_id(1)
    rows = pl.ds(si * RB, RB)
    h2 = h2_ref[...].astype(jnp.bfloat16)
    a = jnp.dot(h2, w1_ref[0], preferred_element_type=jnp.float32)
    b = jnp.dot(h2, w3_ref[0], preferred_element_type=jnp.float32)
    act = (a * jax.nn.sigmoid(a) * b).astype(jnp.bfloat16)
    y = jnp.dot(act, w2_ref[0], preferred_element_type=jnp.float32)
    iota128 = jax.lax.broadcasted_iota(jnp.int32, (RB, 128), 1)
    ge = jnp.sum(jnp.where(iota128 == e, g_ref[...], 0.0), axis=1, keepdims=True)
    contrib = ge * y

    @pl.when(e == 0)
    def _():
        acc_ref[rows, :] = contrib

    @pl.when(e > 0)
    def _():
        acc_ref[rows, :] = acc_ref[rows, :] + contrib

    @pl.when(e == E - 1)
    def _():
        out_ref[...] = acc_ref[rows, :] + x2_ref[...]


def _split_halves(w):
    g = w.shape[-1] // HD
    w3d = w.reshape(*w.shape[:-1], g, HD)
    return jnp.concatenate(
        [w3d[..., :HHD].reshape(*w.shape[:-1], g * HHD),
         w3d[..., HHD:].reshape(*w.shape[:-1], g * HHD)], axis=-1)


def kernel(hidden_states, attention_mask, position_ids, ln1_w, ln1_b, wq, bq,
           wk, bk, wv, bv, wo, bo, ln2_w, ln2_b, w_router, w1, w3, w2):
    x = hidden_states.reshape(S, D)
    pos = position_ids.reshape(S).astype(jnp.float32)
    inv = jnp.asarray(1.0 / (THETA ** (np.arange(0, HD, 2) / HD)), jnp.float32)
    ang = pos[:, None] * inv[None, :]
    cos = jnp.cos(ang)
    sin = jnp.sin(ang)

    wq_p = _split_halves(wq)
    bq_p = _split_halves(bq[None, :])[0]
    wk_p = _split_halves(wk)
    bk_p = _split_halves(bk[None, :])[0]
    wr_pad = jnp.pad(w_router, ((0, 0), (0, 128 - E)))

    row_spec = pl.BlockSpec((RB, D), lambda i: (i, 0))
    full = lambda *shape: pl.BlockSpec(shape, lambda *i: (0,) * len(shape))

    qf, kf, vf = pl.pallas_call(
        _qkv_body,
        grid=(NRB,),
        in_specs=[
            row_spec,
            pl.BlockSpec((RB, HHD), lambda i: (i, 0)),
            pl.BlockSpec((RB, HHD), lambda i: (i, 0)),
            full(D), full(D),
            full(D, H * HD), full(H * HD),
            full(D, KVH * HD), full(KVH * HD),
            full(D, KVH * HD), full(KVH * HD),
        ],
        out_specs=[
            pl.BlockSpec((RB, H * HD), lambda i: (i, 0)),
            pl.BlockSpec((RB, KVH * HD), lambda i: (i, 0)),
            pl.BlockSpec((RB, KVH * HD), lambda i: (i, 0)),
        ],
        out_shape=[
            jax.ShapeDtypeStruct((S, H * HD), jnp.bfloat16),
            jax.ShapeDtypeStruct((S, KVH * HD), jnp.bfloat16),
            jax.ShapeDtypeStruct((S, KVH * HD), jnp.bfloat16),
        ],
        interpret=_INTERPRET,
    )(x, cos, sin, ln1_w, ln1_b, wq_p.astype(jnp.bfloat16),
      bq_p, wk_p.astype(jnp.bfloat16), bk_p, wv.astype(jnp.bfloat16), bv)

    attn = pl.pallas_call(
        _attn_body,
        grid=(NRB,),
        in_specs=[
            pl.BlockSpec((RB, H * HD), lambda i: (i, 0)),
            full(S, KVH * HD),
            full(S, KVH * HD),
        ],
        out_specs=pl.BlockSpec((RB, H * HD), lambda i: (i, 0)),
        out_shape=jax.ShapeDtypeStruct((S, H * HD), jnp.bfloat16),
        interpret=_INTERPRET,
    )(qf, kf, vf)

    x2, h2, gates = pl.pallas_call(
        _post_body,
        grid=(NRB,),
        in_specs=[
            row_spec,
            pl.BlockSpec((RB, H * HD), lambda i: (i, 0)),
            full(H * HD, D), full(D),
            full(D), full(D),
            full(D, 128),
        ],
        out_specs=[row_spec, row_spec, pl.BlockSpec((RB, 128), lambda i: (i, 0))],
        out_shape=[
            jax.ShapeDtypeStruct((S, D), jnp.float32),
            jax.ShapeDtypeStruct((S, D), jnp.float32),
            jax.ShapeDtypeStruct((S, 128), jnp.float32),
        ],
        interpret=_INTERPRET,
    )(x, attn, wo.astype(jnp.bfloat16), bo, ln2_w, ln2_b, wr_pad)

    w1b = w1.astype(jnp.bfloat16)
    w3b = w3.astype(jnp.bfloat16)
    w2b = w2.astype(jnp.bfloat16)

    out = pl.pallas_call(
        _moe_body,
        grid=(E, NRB),
        in_specs=[
            pl.BlockSpec((RB, D), lambda e, s: (s, 0)),
            pl.BlockSpec((RB, 128), lambda e, s: (s, 0)),
            pl.BlockSpec((RB, D), lambda e, s: (s, 0)),
            pl.BlockSpec((1, D, FF), lambda e, s: (e, 0, 0)),
            pl.BlockSpec((1, D, FF), lambda e, s: (e, 0, 0)),
            pl.BlockSpec((1, FF, D), lambda e, s: (e, 0, 0)),
        ],
        out_specs=pl.BlockSpec(
            (RB, D), lambda e, s: (jnp.where(e == E - 1, s, 0), 0)),
        out_shape=jax.ShapeDtypeStruct((S, D), jnp.float32),
        scratch_shapes=[pltpu.VMEM((S, D), jnp.float32)],
        interpret=_INTERPRET,
    )(h2, gates, x2, w1b, w3b, w2b)

    return out.reshape(1, S, D)
